# Initial kernel scaffold; baseline (speedup 1.0000x reference)
import jax
import jax.numpy as jnp
from jax.experimental import pallas as pl


def kernel(z, edge_index, batch_size):
    raise NotImplementedError("write your pallas kernel here")


# speedup vs baseline: 1.3707x; 1.3707x over previous
import functools

import jax
import jax.numpy as jnp
from jax import lax
from jax.experimental import pallas as pl
from jax.experimental.pallas import tpu as pltpu
from jax.experimental.pallas import tpu_sc as plsc

EPS = 1e-9
LANES = 16
NC, NS = 2, 16
NW = NC * NS
CH = 80
GRP = CH // LANES


def _quad16(srows_v, drows_v, e0):
    rowi = e0 + lax.iota(jnp.int32, LANES)
    zero = jnp.zeros((LANES,), jnp.float32)
    dp, na, nb = zero, zero, zero
    for d in range(128):
        coli = jnp.full((LANES,), d, jnp.int32)
        sd = plsc.load_gather(srows_v, [rowi, coli])
        td = plsc.load_gather(drows_v, [rowi, coli])
        if d == 127:
            dp = dp - sd * td
            na = na - sd * sd
            nb = nb - td * td
        else:
            dp = dp + sd * td
            na = na + sd * sd
            nb = nb + td * td
    denom = jnp.maximum(jnp.abs(na * nb), EPS)
    return 1.0 - dp * dp / denom


def _sc_body(z_hbm, pos_hbm, neg_hbm, bs_hbm,
             pos_out, cnt_out, neg_out,
             sidx_v, didx_v, srows_v, drows_v, bs_v,
             accp_v, accc_v, accn_v, sem,
             *, per_tile, neg_per_tile, n_nodes):
    c = lax.axis_index("c")
    s = lax.axis_index("s")
    wid = s * NC + c

    pltpu.sync_copy(bs_hbm, bs_v)
    bs_vec = bs_v[...]
    zero = jnp.zeros((LANES,), jnp.float32)
    accp_v[...] = zero
    accc_v[...] = zero
    accn_v[...] = zero

    def fetch_chunk(src_ref, row_len, base):
        pltpu.sync_copy(src_ref.at[pl.ds(base, CH)], sidx_v)
        pltpu.sync_copy(src_ref.at[pl.ds(row_len + base, CH)], didx_v)
        c1 = pltpu.async_copy(z_hbm.at[sidx_v], srows_v, sem)
        c2 = pltpu.async_copy(z_hbm.at[didx_v], drows_v, sem)
        c1.wait()
        c2.wait()

    def pos_chunk(k, _):
        fetch_chunk(pos_hbm, per_tile * NW, wid * per_tile + k * CH)

        def grp(g, _):
            e0 = g * LANES
            q = _quad16(srows_v, drows_v, e0)
            si = sidx_v[pl.ds(e0, LANES)]
            di = didx_v[pl.ds(e0, LANES)]
            mf = jnp.where((si < bs_vec) & (di < bs_vec), 1.0, 0.0)
            accp_v[...] = accp_v[...] + jnp.minimum(q, 10.0) * mf
            accc_v[...] = accc_v[...] + mf
            return 0

        return lax.fori_loop(0, GRP, grp, 0)

    lax.fori_loop(0, per_tile // CH, pos_chunk, 0)

    def neg_chunk(k, _):
        base = wid * neg_per_tile + k * CH
        fetch_chunk(neg_hbm, neg_per_tile * NW, base)

        def grp(g, _):
            e0 = g * LANES
            q = _quad16(srows_v, drows_v, e0)
            gid = base + e0 + lax.iota(jnp.int32, LANES)
            mf = jnp.where(gid < n_nodes, 1.0, 0.0)
            accn_v[...] = accn_v[...] + jnp.maximum(1.0 - q, 0.0) * mf
            return 0

        return lax.fori_loop(0, GRP, grp, 0)

    lax.fori_loop(0, neg_per_tile // CH, neg_chunk, 0)

    pltpu.sync_copy(accp_v, pos_out.at[wid])
    pltpu.sync_copy(accc_v, cnt_out.at[wid])
    pltpu.sync_copy(accn_v, neg_out.at[wid])


@functools.partial(jax.jit, static_argnums=())
def _uhg_loss_sc(z, edge_index, neg_padded, bs_vec):
    n_nodes, d_model = z.shape
    n_edges = edge_index.shape[0] // 2
    per_tile = n_edges // NW
    neg_per_tile = neg_padded.shape[0] // 2 // NW

    body = functools.partial(
        _sc_body, per_tile=per_tile, neg_per_tile=neg_per_tile,
        n_nodes=n_nodes)
    out_sds = jax.ShapeDtypeStruct((NW, LANES), jnp.float32)
    mesh = plsc.VectorSubcoreMesh(core_axis_name="c", subcore_axis_name="s")
    f = pl.kernel(
        body,
        out_type=(out_sds, out_sds, out_sds),
        mesh=mesh,
        compiler_params=pltpu.CompilerParams(needs_layout_passes=False),
        scratch_types=[
            pltpu.VMEM((CH,), jnp.int32),
            pltpu.VMEM((CH,), jnp.int32),
            pltpu.VMEM((CH, d_model), jnp.float32),
            pltpu.VMEM((CH, d_model), jnp.float32),
            pltpu.VMEM((LANES,), jnp.int32),
            pltpu.VMEM((LANES,), jnp.float32),
            pltpu.VMEM((LANES,), jnp.float32),
            pltpu.VMEM((LANES,), jnp.float32),
            pltpu.SemaphoreType.DMA,
        ],
    )
    return f(z, edge_index, neg_padded, bs_vec)


def kernel(z, edge_index, batch_size):
    n_nodes = z.shape[0]
    neg = jax.random.randint(jax.random.key(42), (2, n_nodes), 0, batch_size,
                             dtype=jnp.int32)
    neg_cap = ((n_nodes + NW * CH - 1) // (NW * CH)) * (NW * CH)
    neg_padded = jnp.pad(neg, ((0, 0), (0, neg_cap - n_nodes)))
    bs_vec = jnp.full((LANES,), batch_size, dtype=jnp.int32)

    pos_s, cnt_s, neg_s = _uhg_loss_sc(
        z, edge_index.reshape(-1), neg_padded.reshape(-1), bs_vec)

    pos_sum = jnp.sum(pos_s)
    count = jnp.sum(cnt_s)
    neg_sum = jnp.sum(neg_s)
    pos_loss = pos_sum / count
    neg_loss = neg_sum / n_nodes
    total = 0.5 * (pos_loss + neg_loss) + 0.01 * pos_loss
    return jnp.clip(total, 0.0, 1000.0)


# speedup vs baseline: 1.7064x; 1.2449x over previous
import functools

import jax
import jax.numpy as jnp
from jax import lax
from jax.experimental import pallas as pl
from jax.experimental.pallas import tpu as pltpu
from jax.experimental.pallas import tpu_sc as plsc

EPS = 1e-9
LANES = 16
NC, NS = 2, 16
NW = NC * NS
CH = 80
GRP = CH // LANES


def _quad16(srows_v, drows_v, e0):
    rowi = e0 + lax.iota(jnp.int32, LANES)
    zero = jnp.zeros((LANES,), jnp.float32)
    dp, na, nb = zero, zero, zero
    for d in range(128):
        coli = jnp.full((LANES,), d, jnp.int32)
        sd = plsc.load_gather(srows_v, [rowi, coli])
        td = plsc.load_gather(drows_v, [rowi, coli])
        if d == 127:
            dp = dp - sd * td
            na = na - sd * sd
            nb = nb - td * td
        else:
            dp = dp + sd * td
            na = na + sd * sd
            nb = nb + td * td
    denom = jnp.maximum(jnp.abs(na * nb), EPS)
    return 1.0 - dp * dp / denom


def _sc_body(z_hbm, pos_hbm, neg_hbm, bs_hbm,
             pos_out, cnt_out, neg_out,
             sidx_v, didx_v, srows0_v, drows0_v, srows1_v, drows1_v, bs_v,
             accp_v, accc_v, accn_v, sem0, sem1,
             *, per_tile, neg_per_tile, n_nodes):
    c = lax.axis_index("c")
    s = lax.axis_index("s")
    wid = s * NC + c
    n_pos_chunks = per_tile // CH
    n_chunks = n_pos_chunks + neg_per_tile // CH
    n_edges = per_tile * NW
    n_neg = neg_per_tile * NW

    pltpu.sync_copy(pos_hbm.at[pl.ds(wid * per_tile, per_tile)],
                    sidx_v.at[pl.ds(0, per_tile)])
    pltpu.sync_copy(pos_hbm.at[pl.ds(n_edges + wid * per_tile, per_tile)],
                    didx_v.at[pl.ds(0, per_tile)])
    pltpu.sync_copy(neg_hbm.at[pl.ds(wid * neg_per_tile, neg_per_tile)],
                    sidx_v.at[pl.ds(per_tile, neg_per_tile)])
    pltpu.sync_copy(neg_hbm.at[pl.ds(n_neg + wid * neg_per_tile, neg_per_tile)],
                    didx_v.at[pl.ds(per_tile, neg_per_tile)])
    pltpu.sync_copy(bs_hbm, bs_v)
    bs_vec = bs_v[...]
    zero = jnp.zeros((LANES,), jnp.float32)
    accp_v[...] = zero
    accc_v[...] = zero
    accn_v[...] = zero

    def issue(k, srows, drows, sem):
        pltpu.async_copy(z_hbm.at[sidx_v.at[pl.ds(k * CH, CH)]], srows, sem)
        pltpu.async_copy(z_hbm.at[didx_v.at[pl.ds(k * CH, CH)]], drows, sem)

    def drain(srows, drows, sem):
        pltpu.make_async_copy(z_hbm.at[pl.ds(0, CH)], srows, sem).wait()
        pltpu.make_async_copy(z_hbm.at[pl.ds(0, CH)], drows, sem).wait()

    def compute(k, srows, drows):
        is_pos = k < n_pos_chunks
        neg_off = wid * neg_per_tile + k * CH - per_tile

        def grp(g, _):
            e0 = g * LANES
            q = _quad16(srows, drows, e0)
            si = sidx_v[pl.ds(k * CH + e0, LANES)]
            di = didx_v[pl.ds(k * CH + e0, LANES)]
            inb = (si < bs_vec) & (di < bs_vec)
            pos_vec = jnp.full((LANES,), is_pos)
            mfp = jnp.where(pos_vec & inb, 1.0, 0.0)
            gid = neg_off + e0 + lax.iota(jnp.int32, LANES)
            mfn = jnp.where((~pos_vec) & (gid < n_nodes), 1.0, 0.0)
            accp_v[...] = accp_v[...] + jnp.minimum(q, 10.0) * mfp
            accc_v[...] = accc_v[...] + mfp
            accn_v[...] = accn_v[...] + jnp.maximum(1.0 - q, 0.0) * mfn
            return 0

        lax.fori_loop(0, GRP, grp, 0)

    issue(0, srows0_v, drows0_v, sem0)

    def pair(j, _):
        k0 = 2 * j
        issue(k0 + 1, srows1_v, drows1_v, sem1)
        drain(srows0_v, drows0_v, sem0)
        compute(k0, srows0_v, drows0_v)
        issue(k0 + 2, srows0_v, drows0_v, sem0)
        drain(srows1_v, drows1_v, sem1)
        compute(k0 + 1, srows1_v, drows1_v)
        return 0

    lax.fori_loop(0, (n_chunks - 1) // 2, pair, 0)
    drain(srows0_v, drows0_v, sem0)
    compute(n_chunks - 1, srows0_v, drows0_v)

    pltpu.sync_copy(accp_v, pos_out.at[wid])
    pltpu.sync_copy(accc_v, cnt_out.at[wid])
    pltpu.sync_copy(accn_v, neg_out.at[wid])


@jax.jit
def _uhg_loss_sc(z, edge_index, neg_padded, bs_vec):
    n_nodes, d_model = z.shape
    n_edges = edge_index.shape[0] // 2
    per_tile = n_edges // NW
    neg_per_tile = neg_padded.shape[0] // 2 // NW

    body = functools.partial(
        _sc_body, per_tile=per_tile, neg_per_tile=neg_per_tile,
        n_nodes=n_nodes)
    out_sds = jax.ShapeDtypeStruct((NW, LANES), jnp.float32)
    mesh = plsc.VectorSubcoreMesh(core_axis_name="c", subcore_axis_name="s")
    f = pl.kernel(
        body,
        out_type=(out_sds, out_sds, out_sds),
        mesh=mesh,
        compiler_params=pltpu.CompilerParams(needs_layout_passes=False),
        scratch_types=[
            pltpu.VMEM((per_tile + neg_per_tile,), jnp.int32),
            pltpu.VMEM((per_tile + neg_per_tile,), jnp.int32),
            pltpu.VMEM((CH, d_model), jnp.float32),
            pltpu.VMEM((CH, d_model), jnp.float32),
            pltpu.VMEM((CH, d_model), jnp.float32),
            pltpu.VMEM((CH, d_model), jnp.float32),
            pltpu.VMEM((LANES,), jnp.int32),
            pltpu.VMEM((LANES,), jnp.float32),
            pltpu.VMEM((LANES,), jnp.float32),
            pltpu.VMEM((LANES,), jnp.float32),
            pltpu.SemaphoreType.DMA,
            pltpu.SemaphoreType.DMA,
        ],
    )
    return f(z, edge_index, neg_padded, bs_vec)


def kernel(z, edge_index, batch_size):
    n_nodes = z.shape[0]
    neg = jax.random.randint(jax.random.key(42), (2, n_nodes), 0, batch_size,
                             dtype=jnp.int32)
    neg_cap = ((n_nodes + NW * CH - 1) // (NW * CH)) * (NW * CH)
    neg_padded = jnp.pad(neg, ((0, 0), (0, neg_cap - n_nodes)))
    bs_vec = jnp.full((LANES,), batch_size, dtype=jnp.int32)

    pos_s, cnt_s, neg_s = _uhg_loss_sc(
        z, edge_index.reshape(-1), neg_padded.reshape(-1), bs_vec)

    pos_sum = jnp.sum(pos_s)
    count = jnp.sum(cnt_s)
    neg_sum = jnp.sum(neg_s)
    pos_loss = pos_sum / count
    neg_loss = neg_sum / n_nodes
    total = 0.5 * (pos_loss + neg_loss) + 0.01 * pos_loss
    return jnp.clip(total, 0.0, 1000.0)


# speedup vs baseline: 7.6571x; 4.4874x over previous
import functools

import jax
import jax.numpy as jnp
from jax import lax
from jax.experimental import pallas as pl
from jax.experimental.pallas import tpu as pltpu
from jax.experimental.pallas import tpu_sc as plsc

EPS = 1e-9
LANES = 16
NC, NS = 2, 16
NW = NC * NS
CH = 80
GRP = CH // LANES


def _quad16(srows_v, drows_v, tab_v, si, di, e0):
    lanes = lax.iota(jnp.int32, LANES)
    rowi = e0 + lanes
    p = jnp.zeros((LANES,), jnp.float32)
    for t in range(128):
        feat = (lanes + t) & 127
        sd = plsc.load_gather(srows_v, [rowi, feat])
        td = plsc.load_gather(drows_v, [rowi, feat])
        p = p + sd * td
    si2 = si * 2
    di2 = di * 2
    ns = plsc.load_gather(tab_v, [si2])
    cs = plsc.load_gather(tab_v, [si2 + 1])
    nd = plsc.load_gather(tab_v, [di2])
    cd = plsc.load_gather(tab_v, [di2 + 1])
    dp = p - 2.0 * cs * cd
    denom = jnp.maximum(jnp.abs(ns * nd), EPS)
    return 1.0 - dp * dp / denom


def _sc_body(z_hbm, tab_hbm, pos_hbm, neg_hbm, bs_hbm,
             pos_out, cnt_out, neg_out,
             sidx_v, didx_v, tab_v,
             srows0_v, drows0_v, srows1_v, drows1_v,
             bs_v, accp_v, accc_v, accn_v, sem0, sem1,
             *, per_tile, neg_per_tile, n_nodes):
    c = lax.axis_index("c")
    s = lax.axis_index("s")
    wid = s * NC + c
    n_pos_chunks = per_tile // CH
    n_chunks = n_pos_chunks + neg_per_tile // CH
    n_edges = per_tile * NW
    n_neg = neg_per_tile * NW

    pltpu.sync_copy(pos_hbm.at[pl.ds(wid * per_tile, per_tile)],
                    sidx_v.at[pl.ds(0, per_tile)])
    pltpu.sync_copy(pos_hbm.at[pl.ds(n_edges + wid * per_tile, per_tile)],
                    didx_v.at[pl.ds(0, per_tile)])
    pltpu.sync_copy(neg_hbm.at[pl.ds(wid * neg_per_tile, neg_per_tile)],
                    sidx_v.at[pl.ds(per_tile, neg_per_tile)])
    pltpu.sync_copy(neg_hbm.at[pl.ds(n_neg + wid * neg_per_tile, neg_per_tile)],
                    didx_v.at[pl.ds(per_tile, neg_per_tile)])
    pltpu.sync_copy(bs_hbm, bs_v)
    pltpu.sync_copy(tab_hbm, tab_v)
    bs_vec = bs_v[...]
    zero = jnp.zeros((LANES,), jnp.float32)
    accp_v[...] = zero
    accc_v[...] = zero
    accn_v[...] = zero

    def issue(k, srows, drows, sem):
        si = sidx_v.at[pl.ds(k * CH, CH)]
        di = didx_v.at[pl.ds(k * CH, CH)]
        pltpu.async_copy(z_hbm.at[si], srows, sem)
        pltpu.async_copy(z_hbm.at[di], drows, sem)

    def drain(srows, drows, sem):
        pltpu.make_async_copy(z_hbm.at[pl.ds(0, CH)], srows, sem).wait()
        pltpu.make_async_copy(z_hbm.at[pl.ds(0, CH)], drows, sem).wait()

    def compute(k, srows, drows):
        is_pos = k < n_pos_chunks
        neg_off = wid * neg_per_tile + k * CH - per_tile

        def grp(g, _):
            e0 = g * LANES
            si = sidx_v[pl.ds(k * CH + e0, LANES)]
            di = didx_v[pl.ds(k * CH + e0, LANES)]
            q = _quad16(srows, drows, tab_v, si, di, e0)
            inb = (si < bs_vec) & (di < bs_vec)
            pos_vec = jnp.full((LANES,), is_pos)
            mfp = jnp.where(pos_vec & inb, 1.0, 0.0)
            gid = neg_off + e0 + lax.iota(jnp.int32, LANES)
            mfn = jnp.where((~pos_vec) & (gid < n_nodes), 1.0, 0.0)
            accp_v[...] = accp_v[...] + jnp.minimum(q, 10.0) * mfp
            accc_v[...] = accc_v[...] + mfp
            accn_v[...] = accn_v[...] + jnp.maximum(1.0 - q, 0.0) * mfn
            return 0

        lax.fori_loop(0, GRP, grp, 0)

    issue(0, srows0_v, drows0_v, sem0)

    def pair(j, _):
        k0 = 2 * j
        issue(k0 + 1, srows1_v, drows1_v, sem1)
        drain(srows0_v, drows0_v, sem0)
        compute(k0, srows0_v, drows0_v)
        issue(k0 + 2, srows0_v, drows0_v, sem0)
        drain(srows1_v, drows1_v, sem1)
        compute(k0 + 1, srows1_v, drows1_v)
        return 0

    lax.fori_loop(0, (n_chunks - 1) // 2, pair, 0)
    drain(srows0_v, drows0_v, sem0)
    compute(n_chunks - 1, srows0_v, drows0_v)

    pltpu.sync_copy(accp_v, pos_out.at[wid])
    pltpu.sync_copy(accc_v, cnt_out.at[wid])
    pltpu.sync_copy(accn_v, neg_out.at[wid])


@jax.jit
def _uhg_loss_sc(z, tab, edge_index, neg_padded, bs_vec):
    n_nodes, d_model = z.shape
    n_edges = edge_index.shape[0] // 2
    per_tile = n_edges // NW
    neg_per_tile = neg_padded.shape[0] // 2 // NW

    body = functools.partial(
        _sc_body, per_tile=per_tile, neg_per_tile=neg_per_tile,
        n_nodes=n_nodes)
    out_sds = jax.ShapeDtypeStruct((NW, LANES), jnp.float32)
    mesh = plsc.VectorSubcoreMesh(core_axis_name="c", subcore_axis_name="s")
    rows_t = pltpu.VMEM((CH, d_model), jnp.float32)
    f = pl.kernel(
        body,
        out_type=(out_sds, out_sds, out_sds),
        mesh=mesh,
        compiler_params=pltpu.CompilerParams(needs_layout_passes=False),
        scratch_types=[
            pltpu.VMEM((per_tile + neg_per_tile,), jnp.int32),
            pltpu.VMEM((per_tile + neg_per_tile,), jnp.int32),
            pltpu.VMEM((n_nodes * 2,), jnp.float32),
            rows_t, rows_t, rows_t, rows_t,
            pltpu.VMEM((LANES,), jnp.int32),
            pltpu.VMEM((LANES,), jnp.float32),
            pltpu.VMEM((LANES,), jnp.float32),
            pltpu.VMEM((LANES,), jnp.float32),
            pltpu.SemaphoreType.DMA,
            pltpu.SemaphoreType.DMA,
        ],
    )
    return f(z, tab, edge_index, neg_padded, bs_vec)


def kernel(z, edge_index, batch_size):
    n_nodes = z.shape[0]
    neg = jax.random.randint(jax.random.key(42), (2, n_nodes), 0, batch_size,
                             dtype=jnp.int32)
    neg_cap = ((n_nodes + NW * CH - 1) // (NW * CH)) * (NW * CH)
    neg_padded = jnp.pad(neg, ((0, 0), (0, neg_cap - n_nodes)))
    bs_vec = jnp.full((LANES,), batch_size, dtype=jnp.int32)
    nt = jnp.sum(z[:, :-1] ** 2, axis=1) - z[:, -1] ** 2
    tab = jnp.stack([nt, z[:, -1]], axis=1).reshape(-1)

    pos_s, cnt_s, neg_s = _uhg_loss_sc(
        z, tab, edge_index.reshape(-1), neg_padded.reshape(-1), bs_vec)

    pos_sum = jnp.sum(pos_s)
    count = jnp.sum(cnt_s)
    neg_sum = jnp.sum(neg_s)
    pos_loss = pos_sum / count
    neg_loss = neg_sum / n_nodes
    total = 0.5 * (pos_loss + neg_loss) + 0.01 * pos_loss
    return jnp.clip(total, 0.0, 1000.0)
